# baseline (speedup 1.0000x reference)
import jax
import jax.numpy as jnp
from jax.experimental import pallas as pl
from jax.experimental.pallas import tpu as pltpu

N = 4096
DIN = 256
DOUT = 256
BM = 256


def _fused_body(x_ref, w_ref, s0_ref, s1_ref, mean_ref, var_ref, a_ref, b_ref):
    i = pl.program_id(0)

    @pl.when(i == 0)
    def _phase_a():
        enc = jnp.dot(x_ref[...], w_ref[...], preferred_element_type=jnp.float32)
        m = jnp.maximum(enc[:, :DOUT], 0.0)
        v = jnp.maximum(enc[:, DOUT:], 0.0)
        nw = jnp.exp(-v)
        a_ref[...] = (m * nw).astype(jnp.bfloat16)
        b_ref[...] = (v * nw * nw).astype(jnp.bfloat16)

    s0 = s0_ref[...].astype(jnp.bfloat16)
    s1 = s1_ref[...].astype(jnp.bfloat16)
    mo = jnp.dot(s0, a_ref[...], preferred_element_type=jnp.float32)
    vo = jnp.dot(s1, b_ref[...], preferred_element_type=jnp.float32)
    mean_ref[...] = jnp.maximum(mo, 0.0)
    var_ref[...] = jnp.where(vo > 0.0, vo, jnp.exp(jnp.minimum(vo, 0.0)) - 1.0) + (1.0 + 1e-14)


def kernel(x, support0, support1, W):
    grid = (N // BM,)
    out_shape = (
        jax.ShapeDtypeStruct((N, DOUT), jnp.float32),
        jax.ShapeDtypeStruct((N, DOUT), jnp.float32),
    )
    mean_out, var_out = pl.pallas_call(
        _fused_body,
        grid=grid,
        in_specs=[
            pl.BlockSpec((N, DIN), lambda i: (0, 0)),
            pl.BlockSpec((DIN, 2 * DOUT), lambda i: (0, 0)),
            pl.BlockSpec((BM, N), lambda i: (i, 0)),
            pl.BlockSpec((BM, N), lambda i: (i, 0)),
        ],
        out_specs=[
            pl.BlockSpec((BM, DOUT), lambda i: (i, 0)),
            pl.BlockSpec((BM, DOUT), lambda i: (i, 0)),
        ],
        out_shape=out_shape,
        scratch_shapes=[
            pltpu.VMEM((N, DOUT), jnp.bfloat16),
            pltpu.VMEM((N, DOUT), jnp.bfloat16),
        ],
        compiler_params=pltpu.CompilerParams(
            dimension_semantics=("arbitrary",),
        ),
    )(x, W, support0, support1)
    return (mean_out, var_out)
